# baseline (speedup 1.0000x reference)
import functools

import jax
import jax.numpy as jnp
from jax import lax
from jax.experimental import pallas as pl
from jax.experimental.pallas import tpu as pltpu
from jax.experimental.pallas import tpu_sc as plsc

_VOCAB = 1000000
_EMBED = 64
_BATCH = 4096
_SEQ = 200
_NC, _NS = 2, 16
_NW = _NC * _NS
_BATW = _BATCH // _NW
_SPLITS = ((0, 104), (104, 96))
_NBUF = 4
_T = _BATW // _NBUF

_mesh = plsc.VectorSubcoreMesh(core_axis_name="c", subcore_axis_name="s")


@functools.partial(
    pl.kernel,
    mesh=_mesh,
    compiler_params=pltpu.CompilerParams(use_tc_tiling_on_sc=False),
    out_type=jax.ShapeDtypeStruct((_BATCH, _SEQ, _EMBED), jnp.float32),
    scratch_types=[
        pltpu.VMEM((_BATW, _SEQ), jnp.int32),
        pltpu.VMEM((_NBUF, 1, _SEQ, _EMBED), jnp.float32),
    ] + [pltpu.SemaphoreType.DMA] * (2 * _NBUF),
)
def _gather_sc(idx_hbm, table_hbm, out_hbm, idx_v, rows_v, *sems):
    gsem = sems[:_NBUF]
    ssem = sems[_NBUF:]
    wid = lax.axis_index("s") * _NC + lax.axis_index("c")
    base = wid * _BATW
    pltpu.sync_copy(idx_hbm.at[pl.ds(base, _BATW)], idx_v)

    def issue_gather(s, b):
        for off, size in _SPLITS:
            pltpu.async_copy(
                table_hbm.at[idx_v.at[s, pl.ds(off, size)]],
                rows_v.at[b, 0, pl.ds(off, size)],
                gsem[b],
            )

    def wait_gather(b):
        for off, size in _SPLITS:
            pltpu.make_async_copy(
                table_hbm.at[idx_v.at[0, pl.ds(off, size)]],
                rows_v.at[b, 0, pl.ds(off, size)],
                gsem[b],
            ).wait()

    def issue_scatter(s, b):
        pltpu.async_copy(rows_v.at[b], out_hbm.at[pl.ds(base + s, 1)], ssem[b])

    def wait_scatter(b):
        pltpu.make_async_copy(
            rows_v.at[b], out_hbm.at[pl.ds(base, 1)], ssem[b]
        ).wait()

    def step(s, b, do_issue, do_wait_prev):
        pb = (b - 1) % _NBUF
        wait_gather(b)
        issue_scatter(s, b)
        if do_wait_prev:
            wait_scatter(pb)
        if do_issue:
            issue_gather(s + _NBUF - 1, pb)

    for b in range(_NBUF - 1):
        issue_gather(b, b)
    step(0, 0, True, False)
    for b in range(1, _NBUF):
        step(b, b, True, True)

    def body(t, carry):
        for b in range(_NBUF):
            step(t * _NBUF + b, b, True, True)
        return carry

    lax.fori_loop(1, _T - 1, body, 0)

    s0 = (_T - 1) * _NBUF
    step(s0, 0, True, True)
    for b in range(1, _NBUF):
        step(s0 + b, b, False, True)
    wait_scatter(_NBUF - 1)


def kernel(x, table):
    return _gather_sc(x.astype(jnp.int32), table)
